# baseline (speedup 1.0000x reference)
import functools

import jax
import jax.numpy as jnp
from jax import lax
from jax.experimental import pallas as pl
from jax.experimental.pallas import tpu as pltpu
from jax.experimental.pallas import tpu_sc as plsc

_NC = 2
_NS = 16
_L = 16


def _prep1_body(x_ref, w_ref, al_ref, ar_ref, g_ref, er_ref, cm_ref):
    feat = jnp.dot(x_ref[...], w_ref[...], preferred_element_type=jnp.float32)
    el = jnp.dot(feat, al_ref[...], preferred_element_type=jnp.float32)
    er = jnp.dot(feat, ar_ref[...], preferred_element_type=jnp.float32)
    c = jnp.max(el, axis=0, keepdims=True) + jnp.max(er, axis=0, keepdims=True)
    cm = jnp.where(c > 0, c, 0.2 * c)
    n, h = el.shape
    fw = feat.shape[1] // _NC
    hh = h // _NC
    halves_g = []
    halves_e = []
    halves_c = []
    for cc in range(_NC):
        halves_g.append(jnp.concatenate(
            [feat[:, cc * fw:(cc + 1) * fw],
             el[:, cc * hh:(cc + 1) * hh],
             jnp.zeros((n, _L - hh), jnp.float32)], axis=1))
        halves_e.append(jnp.concatenate(
            [er[:, cc * hh:(cc + 1) * hh],
             jnp.full((n, _L - hh), -60.0, jnp.float32)], axis=1))
        halves_c.append(jnp.concatenate(
            [cm[:, cc * hh:(cc + 1) * hh],
             jnp.zeros((1, _L - hh), jnp.float32)], axis=1))
    g_ref[...] = jnp.concatenate(halves_g, axis=0)
    er_ref[...] = jnp.concatenate(halves_e, axis=0)
    cm_ref[...] = jnp.concatenate(halves_c, axis=0)


def _mid_body(n_nodes, o1_ref, w2_ref, al2_ref, ar2_ref, s1_ref,
              g2_ref, er2_ref, cm2_ref):
    d = w2_ref.shape[0]
    h = s1_ref.shape[0]
    fw = d // _NC
    hh = h // _NC
    a0 = o1_ref[0]
    a1 = o1_ref[1]
    num = jnp.concatenate([a0[:n_nodes, :fw], a1[:n_nodes, :fw]], axis=1)
    den = jnp.concatenate([a0[:n_nodes, fw:fw + hh],
                           a1[:n_nodes, fw:fw + hh]], axis=1)
    den_w = jnp.dot(den, s1_ref[...], preferred_element_type=jnp.float32)
    q = num / (den_w + 1e-9)
    h1 = jnp.where(q > 0, q, jnp.exp(q) - 1.0)
    feat2 = jnp.dot(h1, w2_ref[...], preferred_element_type=jnp.float32)
    el2 = jnp.dot(feat2, al2_ref[...], preferred_element_type=jnp.float32)
    er2 = jnp.dot(feat2, ar2_ref[...], preferred_element_type=jnp.float32)
    c2 = jnp.max(el2, axis=0, keepdims=True) + jnp.max(er2, axis=0,
                                                       keepdims=True)
    cm2 = jnp.where(c2 > 0, c2, 0.2 * c2)
    h2 = al2_ref.shape[1]
    g2_ref[...] = jnp.concatenate(
        [feat2, el2, jnp.zeros((n_nodes, _L - h2), jnp.float32)], axis=1)
    er2_ref[...] = jnp.concatenate(
        [er2, jnp.full((n_nodes, _L - h2), -60.0, jnp.float32)], axis=1)
    cm2_row = jnp.concatenate(
        [cm2, jnp.zeros((1, _L - h2), jnp.float32)], axis=1)
    cm2_ref[...] = jnp.concatenate([cm2_row] * _NC, axis=0)


def _fin_body(np_half, o2_ref, out_ref):
    n, d = out_ref.shape
    a0 = o2_ref[0]
    a1 = o2_ref[1]
    top = a0[:np_half, :d] / (a0[:np_half, d:d + 1] + 1e-9)
    rest = n - np_half
    bot = a1[:rest, :d] / (a1[:rest, d:d + 1] + 1e-9)
    out_ref[...] = jnp.concatenate([top, bot], axis=0)


def _edge_pass(gtab, ertab, cmtab, src, dst, n_nodes, hh, dim, split_heads,
               np_half=0):
    e_total = src.shape[0]
    ew = e_total // _NS
    nb = 2
    nchunk = ew // 128
    nsuper = nchunk // nb
    rem = nchunk - nsuper * nb
    tail = ew - nchunk * 128
    np_rows = -(-n_nodes // (_NS * 8)) * (_NS * 8)
    acc_rows = np_rows if split_heads else np_half + 128
    rpt = acc_rows // _NS
    gw = dim * hh + _L
    fd = dim * hh
    zrows = 8
    assert rpt % zrows == 0 and tail % _L == 0

    mesh = plsc.VectorSubcoreMesh(core_axis_name="c", subcore_axis_name="s",
                                  num_cores=_NC, num_subcores=_NS)

    def body(g_hbm, er_hbm, cm_hbm, src_hbm, dst_hbm, o_hbm, *scr):
        svs = scr[0:nb]
        dvs = scr[nb:2 * nb]
        sofs = scr[2 * nb:3 * nb]
        dofs = scr[3 * nb:4 * nb]
        gbs = scr[4 * nb:5 * nb]
        ebs = scr[5 * nb:6 * nb]
        obs = scr[6 * nb:7 * nb]
        gsems = scr[7 * nb:8 * nb]
        ssems = scr[8 * nb:9 * nb]
        (srcv2, dstv2, sof2, dof2, gbuf2, erbuf2, obuf2, zbuf, cmbuf,
         ashr) = scr[9 * nb:]

        cc = lax.axis_index("c")
        ss = lax.axis_index("s")
        row0 = ss * rpt
        rowoff = cc * n_nodes
        nodeoff = cc * np_half
        pltpu.sync_copy(cm_hbm.at[cc], cmbuf)

        zero = jnp.zeros((_L,), jnp.float32)
        for r in range(zrows):
            for j in range(gw // _L):
                zbuf[r, pl.ds(_L * j, _L)] = zero
        for b in range(rpt // zrows):
            pltpu.sync_copy(zbuf, ashr.at[pl.ds(row0 + zrows * b, zrows)])
        plsc.subcore_barrier()

        ew0 = ss * ew
        cmv = cmbuf[:]

        def load_idx(off, sv, dv, sof, dof, csz):
            off = pl.multiple_of(off, 8)
            pltpu.sync_copy(src_hbm.at[pl.ds(off, csz)], sv)
            pltpu.sync_copy(dst_hbm.at[pl.ds(off, csz)], dv)
            for g in range(csz // _L):
                if split_heads:
                    sof[pl.ds(_L * g, _L)] = sv[pl.ds(_L * g, _L)] + rowoff
                    dof[pl.ds(_L * g, _L)] = dv[pl.ds(_L * g, _L)] + rowoff
                else:
                    dvv = dv[pl.ds(_L * g, _L)]
                    ld = dvv - nodeoff
                    ok = (ld >= 0) & (ld < np_half)
                    dof[pl.ds(_L * g, _L)] = jnp.where(
                        ok, ld, np_half + (dvv & 7))

        def start_gathers(sv, dv, sof, dof, gb, eb, sem):
            gi = sof if split_heads else sv
            ei = dof if split_heads else dv
            d1 = pltpu.async_copy(g_hbm.at[gi], gb, sem)
            d2 = pltpu.async_copy(er_hbm.at[ei], eb, sem)
            return d1, d2

        def scatter_idx(dv, dof):
            return dv if split_heads else dof

        def compute(gb, eb, ob, csz):
            def ebody(i, carry):
                elv = gb[i, pl.ds(fd, _L)]
                erv = eb[i, :]
                s = elv + erv
                e = jnp.where(s > 0, s, 0.2 * s) - cmv
                ee = jnp.exp(e)
                ob[i, pl.ds(fd, _L)] = ee
                for h in range(hh):
                    ob[i, pl.ds(dim * h, _L)] = (
                        gb[i, pl.ds(dim * h, _L)] * ee[h])
                return carry

            lax.fori_loop(0, csz, ebody, 0, unroll=4)

        def super_body(q, carry):
            base = ew0 + 128 * nb * q
            ds1 = []
            for b in range(nb):
                load_idx(base + 128 * b, svs[b], dvs[b], sofs[b], dofs[b], 128)
                ds1.append(start_gathers(svs[b], dvs[b], sofs[b], dofs[b],
                                         gbs[b], ebs[b], gsems[b]))
            ds2 = []
            for b in range(nb):
                ds1[b][0].wait()
                ds1[b][1].wait()
                compute(gbs[b], ebs[b], obs[b], 128)
                ds2.append(pltpu.async_copy(
                    obs[b], ashr.at[scatter_idx(dvs[b], dofs[b])],
                    ssems[b], add=True))
            for b in range(nb):
                ds2[b].wait()
            return carry

        lax.fori_loop(0, nsuper, super_body, 0)

        def sync_chunk(off, sv, dv, sof, dof, gb, eb, ob, sem, csz):
            load_idx(off, sv, dv, sof, dof, csz)
            d1, d2 = start_gathers(sv, dv, sof, dof, gb, eb, sem)
            d1.wait()
            d2.wait()
            compute(gb, eb, ob, csz)
            pltpu.sync_copy(ob, ashr.at[scatter_idx(dv, dof)], add=True)

        for b in range(rem):
            sync_chunk(ew0 + 128 * (nsuper * nb + b), svs[b], dvs[b],
                       sofs[b], dofs[b], gbs[b], ebs[b], obs[b], gsems[b],
                       128)
        if tail:
            sync_chunk(ew0 + 128 * nchunk, srcv2, dstv2, sof2, dof2,
                       gbuf2, erbuf2, obuf2, gsems[0], tail)

        plsc.subcore_barrier()
        pltpu.sync_copy(ashr.at[pl.ds(row0, rpt)],
                        o_hbm.at[cc, pl.ds(row0, rpt)])

    scratch = (
        [pltpu.VMEM((128,), jnp.int32) for _ in range(nb)]
        + [pltpu.VMEM((128,), jnp.int32) for _ in range(nb)]
        + [pltpu.VMEM((128,), jnp.int32) for _ in range(nb)]
        + [pltpu.VMEM((128,), jnp.int32) for _ in range(nb)]
        + [pltpu.VMEM((128, gw), jnp.float32) for _ in range(nb)]
        + [pltpu.VMEM((128, _L), jnp.float32) for _ in range(nb)]
        + [pltpu.VMEM((128, gw), jnp.float32) for _ in range(nb)]
        + [pltpu.SemaphoreType.DMA for _ in range(nb)]
        + [pltpu.SemaphoreType.DMA for _ in range(nb)]
        + [pltpu.VMEM((tail or 8,), jnp.int32),
           pltpu.VMEM((tail or 8,), jnp.int32),
           pltpu.VMEM((tail or 8,), jnp.int32),
           pltpu.VMEM((tail or 8,), jnp.int32),
           pltpu.VMEM((tail or 8, gw), jnp.float32),
           pltpu.VMEM((tail or 8, _L), jnp.float32),
           pltpu.VMEM((tail or 8, gw), jnp.float32),
           pltpu.VMEM((zrows, gw), jnp.float32),
           pltpu.VMEM((_L,), jnp.float32),
           pltpu.VMEM_SHARED((acc_rows, gw), jnp.float32)]
    )
    out_type = jax.ShapeDtypeStruct((_NC, acc_rows, gw), jnp.float32)
    return pl.kernel(
        body, out_type=out_type, mesh=mesh, scratch_types=scratch,
        compiler_params=pltpu.CompilerParams(use_tc_tiling_on_sc=False),
    )(gtab, ertab, cmtab, src, dst)


def kernel(x, edge_index, W1, attn_l1, attn_r1, W2, attn_l2, attn_r2):
    n, _ = x.shape
    h1, hid = attn_l1.shape
    h2, out_d = attn_l2.shape
    src = edge_index[0].astype(jnp.int32)
    dst = edge_index[1].astype(jnp.int32)

    al1 = (jnp.eye(h1, dtype=jnp.float32)[:, None, :]
           * attn_l1[:, :, None]).reshape(h1 * hid, h1)
    ar1 = (jnp.eye(h1, dtype=jnp.float32)[:, None, :]
           * attn_r1[:, :, None]).reshape(h1 * hid, h1)
    al2 = (jnp.eye(h2, dtype=jnp.float32)[:, None, :]
           * attn_l2[:, :, None]).reshape(h2 * out_d, h2)
    ar2 = (jnp.eye(h2, dtype=jnp.float32)[:, None, :]
           * attn_r2[:, :, None]).reshape(h2 * out_d, h2)
    s1 = jnp.repeat(jnp.eye(h1, dtype=jnp.float32), hid, axis=1)

    gw1 = (h1 // _NC) * hid + _L
    g1, er1, cm1 = pl.pallas_call(
        _prep1_body,
        out_shape=[jax.ShapeDtypeStruct((_NC * n, gw1), jnp.float32),
                   jax.ShapeDtypeStruct((_NC * n, _L), jnp.float32),
                   jax.ShapeDtypeStruct((_NC, _L), jnp.float32)],
    )(x, W1, al1, ar1)

    o1 = _edge_pass(g1, er1, cm1, src, dst, n, h1 // _NC, hid, True)

    g2, er2, cm2 = pl.pallas_call(
        functools.partial(_mid_body, n),
        out_shape=[jax.ShapeDtypeStruct((n, h2 * out_d + _L), jnp.float32),
                   jax.ShapeDtypeStruct((n, _L), jnp.float32),
                   jax.ShapeDtypeStruct((_NC, _L), jnp.float32)],
    )(o1, W2, al2, ar2, s1)

    np_half = -(-(n // 2) // 128) * 128
    o2 = _edge_pass(g2, er2, cm2, src, dst, n, h2, out_d, False,
                    np_half=np_half)

    out = pl.pallas_call(
        functools.partial(_fin_body, np_half),
        out_shape=jax.ShapeDtypeStruct((n, out_d), jnp.float32),
    )(o2)
    return out


# speedup vs baseline: 1.2437x; 1.2437x over previous
import functools

import jax
import jax.numpy as jnp
from jax import lax
from jax.experimental import pallas as pl
from jax.experimental.pallas import tpu as pltpu
from jax.experimental.pallas import tpu_sc as plsc

_NC = 2
_NS = 16
_L = 16


def _prep1_body(x_ref, w_ref, al_ref, ar_ref, g_ref, er_ref, cm_ref):
    feat = jnp.dot(x_ref[...], w_ref[...], preferred_element_type=jnp.float32)
    el = jnp.dot(feat, al_ref[...], preferred_element_type=jnp.float32)
    er = jnp.dot(feat, ar_ref[...], preferred_element_type=jnp.float32)
    c = jnp.max(el, axis=0, keepdims=True) + jnp.max(er, axis=0, keepdims=True)
    cm = jnp.where(c > 0, c, 0.2 * c)
    n, h = el.shape
    fw = feat.shape[1] // _NC
    hh = h // _NC
    halves_g = []
    halves_e = []
    halves_c = []
    for cc in range(_NC):
        halves_g.append(jnp.concatenate(
            [feat[:, cc * fw:(cc + 1) * fw],
             el[:, cc * hh:(cc + 1) * hh],
             jnp.zeros((n, _L - hh), jnp.float32)], axis=1))
        halves_e.append(jnp.concatenate(
            [er[:, cc * hh:(cc + 1) * hh],
             jnp.full((n, _L - hh), -60.0, jnp.float32)], axis=1))
        halves_c.append(jnp.concatenate(
            [cm[:, cc * hh:(cc + 1) * hh],
             jnp.zeros((1, _L - hh), jnp.float32)], axis=1))
    g_ref[...] = jnp.concatenate(halves_g, axis=0)
    er_ref[...] = jnp.concatenate(
        halves_e + [jnp.zeros((_L, _L), jnp.float32)], axis=0)
    cm_ref[...] = jnp.concatenate(halves_c, axis=0)


def _mid_body(n_nodes, o1_ref, w2_ref, al2_ref, ar2_ref, s1_ref,
              g2_ref, er2_ref, cm2_ref):
    d = w2_ref.shape[0]
    h = s1_ref.shape[0]
    fw = d // _NC
    hh = h // _NC
    a0 = o1_ref[0]
    a1 = o1_ref[1]
    num = jnp.concatenate([a0[:n_nodes, :fw], a1[:n_nodes, :fw]], axis=1)
    den = jnp.concatenate([a0[:n_nodes, fw:fw + hh],
                           a1[:n_nodes, fw:fw + hh]], axis=1)
    den_w = jnp.dot(den, s1_ref[...], preferred_element_type=jnp.float32)
    q = num / (den_w + 1e-9)
    h1 = jnp.where(q > 0, q, jnp.exp(q) - 1.0)
    feat2 = jnp.dot(h1, w2_ref[...], preferred_element_type=jnp.float32)
    el2 = jnp.dot(feat2, al2_ref[...], preferred_element_type=jnp.float32)
    er2 = jnp.dot(feat2, ar2_ref[...], preferred_element_type=jnp.float32)
    c2 = jnp.max(el2, axis=0, keepdims=True) + jnp.max(er2, axis=0,
                                                       keepdims=True)
    cm2 = jnp.where(c2 > 0, c2, 0.2 * c2)
    h2 = al2_ref.shape[1]
    g2_ref[...] = jnp.concatenate(
        [feat2, el2, jnp.zeros((n_nodes, _L - h2), jnp.float32)], axis=1)
    er2_main = jnp.concatenate(
        [er2, jnp.full((n_nodes, _L - h2), -60.0, jnp.float32)], axis=1)
    pad_rows = er2_ref.shape[0] - n_nodes
    er2_ref[...] = jnp.concatenate(
        [er2_main, jnp.zeros((pad_rows, _L), jnp.float32)], axis=0)
    cm2_row = jnp.concatenate(
        [cm2, jnp.zeros((1, _L - h2), jnp.float32)], axis=1)
    cm2_ref[...] = jnp.concatenate([cm2_row] * _NC, axis=0)


def _fin_body(o2_ref, out_ref):
    a = o2_ref[0] + o2_ref[1]
    n, d = out_ref.shape
    num = a[:n, :d]
    den = a[:n, d:d + 1]
    out_ref[...] = num / (den + 1e-9)


def _edge_pass(gtab, ertab, cmtab, srcp, dstp, n_nodes, hh, dim,
               split_heads):
    rt = srcp.shape[0]
    nwork = _NS if split_heads else _NC * _NS
    rti = rt // nwork
    bch = 4
    nsuper = rti // bch
    assert rti % bch == 0
    np_rows = -(-n_nodes // (_NS * 8)) * (_NS * 8)
    acc_rows = np_rows
    rpt = acc_rows // _NS
    gw = dim * hh + _L
    fd = dim * hh
    zrows = 8
    assert rpt % zrows == 0

    mesh = plsc.VectorSubcoreMesh(core_axis_name="c", subcore_axis_name="s",
                                  num_cores=_NC, num_subcores=_NS)

    def body(g_hbm, er_hbm, cm_hbm, src_hbm, dst_hbm, o_hbm,
             srcall, dstall, eidxb, gb, eb, zbuf, cmbuf, ashr):
        cc = lax.axis_index("c")
        ss = lax.axis_index("s")
        row0 = ss * rpt
        wid = ss if split_heads else ss * _NC + cc
        pltpu.sync_copy(cm_hbm.at[cc], cmbuf)
        pltpu.sync_copy(src_hbm.at[pl.ds(wid * rti, rti)], srcall)
        pltpu.sync_copy(dst_hbm.at[pl.ds(wid * rti, rti)], dstall)

        if split_heads:
            rowoff = cc * n_nodes

            def trow(r, carry):
                for g in range(8):
                    srcall[r, pl.ds(_L * g, _L)] = (
                        srcall[r, pl.ds(_L * g, _L)] + rowoff)
                return carry

            lax.fori_loop(0, rti, trow, 0, unroll=4)

        zero = jnp.zeros((_L,), jnp.float32)
        for r in range(zrows):
            for j in range(gw // _L):
                zbuf[r, pl.ds(_L * j, _L)] = zero
        for b in range(rpt // zrows):
            pltpu.sync_copy(zbuf, ashr.at[pl.ds(row0 + zrows * b, zrows)])
        plsc.subcore_barrier()

        cmv = cmbuf[:]
        rowoff_e = cc * n_nodes

        def super_body(k, carry):
            if split_heads:
                for g in range(8):
                    eidxb[pl.ds(_L * g, _L)] = (
                        dstall[k, pl.ds(_L * g, _L)] + rowoff_e)
                eref = eidxb
            else:
                eref = dstall.at[k]
            pltpu.sync_copy(g_hbm.at[srcall.at[k]], gb)
            pltpu.sync_copy(er_hbm.at[eref], eb)

            def ebody(i, carry2):
                elv = gb[i, pl.ds(fd, _L)]
                erv = eb[i, :]
                s = elv + erv
                e = jnp.where(s > 0, s, 0.2 * s) - cmv
                ee = jnp.exp(e)
                for h in range(hh):
                    gb[i, pl.ds(dim * h, _L)] = (
                        gb[i, pl.ds(dim * h, _L)] * ee[h])
                gb[i, pl.ds(fd, _L)] = ee
                return carry2

            lax.fori_loop(0, 128, ebody, 0, unroll=4)

            pltpu.sync_copy(gb, ashr.at[dstall.at[k]], add=True)
            return carry

        lax.fori_loop(0, rti, super_body, 0)

        plsc.subcore_barrier()
        pltpu.sync_copy(ashr.at[pl.ds(row0, rpt)],
                        o_hbm.at[cc, pl.ds(row0, rpt)])

    scratch = [
        pltpu.VMEM((rti, 128), jnp.int32),
        pltpu.VMEM((rti, 128), jnp.int32),
        pltpu.VMEM((128,), jnp.int32),
        pltpu.VMEM((128, gw), jnp.float32),
        pltpu.VMEM((128, _L), jnp.float32),
        pltpu.VMEM((zrows, gw), jnp.float32),
        pltpu.VMEM((_L,), jnp.float32),
        pltpu.VMEM_SHARED((acc_rows, gw), jnp.float32),
    ]
    out_type = jax.ShapeDtypeStruct((_NC, acc_rows, gw), jnp.float32)
    return pl.kernel(
        body, out_type=out_type, mesh=mesh, scratch_types=scratch,
        compiler_params=pltpu.CompilerParams(use_tc_tiling_on_sc=False),
    )(gtab, ertab, cmtab, srcp, dstp)


def kernel(x, edge_index, W1, attn_l1, attn_r1, W2, attn_l2, attn_r2):
    n, _ = x.shape
    h1, hid = attn_l1.shape
    h2, out_d = attn_l2.shape
    src = edge_index[0].astype(jnp.int32)
    dst = edge_index[1].astype(jnp.int32)

    e = src.shape[0]
    e_pad = -(-e // (_NS * 128 * 8)) * (_NS * 128 * 8)
    if e_pad > e:
        ii = jnp.arange(e_pad - e, dtype=jnp.int32)
        src = jnp.concatenate([src, ii % 1024])
        dst = jnp.concatenate([dst, n + (ii % 8)])
    src2d = src.reshape(e_pad // 128, 128)
    dst2d = dst.reshape(e_pad // 128, 128)

    al1 = (jnp.eye(h1, dtype=jnp.float32)[:, None, :]
           * attn_l1[:, :, None]).reshape(h1 * hid, h1)
    ar1 = (jnp.eye(h1, dtype=jnp.float32)[:, None, :]
           * attn_r1[:, :, None]).reshape(h1 * hid, h1)
    al2 = (jnp.eye(h2, dtype=jnp.float32)[:, None, :]
           * attn_l2[:, :, None]).reshape(h2 * out_d, h2)
    ar2 = (jnp.eye(h2, dtype=jnp.float32)[:, None, :]
           * attn_r2[:, :, None]).reshape(h2 * out_d, h2)
    s1 = jnp.repeat(jnp.eye(h1, dtype=jnp.float32), hid, axis=1)

    np_rows = -(-n // (_NS * 8)) * (_NS * 8)
    gw1 = (h1 // _NC) * hid + _L
    g1, er1, cm1 = pl.pallas_call(
        _prep1_body,
        out_shape=[jax.ShapeDtypeStruct((_NC * n, gw1), jnp.float32),
                   jax.ShapeDtypeStruct((_NC * n + _L, _L), jnp.float32),
                   jax.ShapeDtypeStruct((_NC, _L), jnp.float32)],
    )(x, W1, al1, ar1)

    o1 = _edge_pass(g1, er1, cm1, src2d, dst2d, n, h1 // _NC, hid, True)

    g2, er2, cm2 = pl.pallas_call(
        functools.partial(_mid_body, n),
        out_shape=[jax.ShapeDtypeStruct((n, h2 * out_d + _L), jnp.float32),
                   jax.ShapeDtypeStruct((np_rows, _L), jnp.float32),
                   jax.ShapeDtypeStruct((_NC, _L), jnp.float32)],
    )(o1, W2, al2, ar2, s1)

    o2 = _edge_pass(g2, er2, cm2, src2d, dst2d, n, h2, out_d, False)

    out = pl.pallas_call(
        _fin_body,
        out_shape=jax.ShapeDtypeStruct((n, out_d), jnp.float32),
    )(o2)
    return out


# speedup vs baseline: 1.5229x; 1.2245x over previous
import functools

import jax
import jax.numpy as jnp
from jax import lax
from jax.experimental import pallas as pl
from jax.experimental.pallas import tpu as pltpu
from jax.experimental.pallas import tpu_sc as plsc

_NC = 2
_NS = 16
_L = 16


def _prep1_body(x_ref, w_ref, al_ref, ar_ref, g_ref, er_ref, cm_ref):
    feat = jnp.dot(x_ref[...], w_ref[...], preferred_element_type=jnp.float32)
    el = jnp.dot(feat, al_ref[...], preferred_element_type=jnp.float32)
    er = jnp.dot(feat, ar_ref[...], preferred_element_type=jnp.float32)
    c = jnp.max(el, axis=0, keepdims=True) + jnp.max(er, axis=0, keepdims=True)
    cm = jnp.where(c > 0, c, 0.2 * c)
    n, h = el.shape
    fw = feat.shape[1] // _NC
    hh = h // _NC
    halves_g = []
    halves_e = []
    halves_c = []
    for cc in range(_NC):
        halves_g.append(jnp.concatenate(
            [feat[:, cc * fw:(cc + 1) * fw],
             el[:, cc * hh:(cc + 1) * hh],
             jnp.zeros((n, _L - hh), jnp.float32)], axis=1))
        halves_e.append(jnp.concatenate(
            [er[:, cc * hh:(cc + 1) * hh],
             jnp.full((n, _L - hh), -60.0, jnp.float32)], axis=1))
        halves_c.append(jnp.concatenate(
            [cm[:, cc * hh:(cc + 1) * hh],
             jnp.zeros((1, _L - hh), jnp.float32)], axis=1))
    g_ref[...] = jnp.concatenate(halves_g, axis=0)
    np_rows = er_ref.shape[1]
    for cc in range(_NC):
        er_ref[cc] = jnp.concatenate(
            [halves_e[cc],
             jnp.zeros((np_rows - n, _L), jnp.float32)], axis=0)
    cm_ref[...] = jnp.concatenate(halves_c, axis=0)


def _mid_body(n_nodes, o1_ref, w2_ref, al2_ref, ar2_ref, s1_ref,
              g2_ref, er2_ref, cm2_ref):
    d = w2_ref.shape[0]
    h = s1_ref.shape[0]
    fw = d // _NC
    hh = h // _NC
    a0 = o1_ref[0]
    a1 = o1_ref[1]
    num = jnp.concatenate([a0[:n_nodes, :fw], a1[:n_nodes, :fw]], axis=1)
    den = jnp.concatenate([a0[:n_nodes, fw:fw + hh],
                           a1[:n_nodes, fw:fw + hh]], axis=1)
    den_w = jnp.dot(den, s1_ref[...], preferred_element_type=jnp.float32)
    q = num / (den_w + 1e-9)
    h1 = jnp.where(q > 0, q, jnp.exp(q) - 1.0)
    feat2 = jnp.dot(h1, w2_ref[...], preferred_element_type=jnp.float32)
    d2 = feat2.shape[1]
    el2 = jnp.dot(feat2, al2_ref[...], preferred_element_type=jnp.float32)
    er2 = jnp.dot(feat2, ar2_ref[...], preferred_element_type=jnp.float32)
    c2 = jnp.max(el2, axis=0, keepdims=True) + jnp.max(er2, axis=0,
                                                       keepdims=True)
    cm2 = jnp.where(c2 > 0, c2, 0.2 * c2)
    h2 = al2_ref.shape[1]
    gw2 = g2_ref.shape[1]
    np_rows = er2_ref.shape[1]
    g2_main = jnp.concatenate(
        [feat2, el2, jnp.zeros((n_nodes, gw2 - d2 - h2), jnp.float32)],
        axis=1)
    g2_ref[...] = jnp.concatenate(
        [g2_main, jnp.zeros((np_rows - n_nodes, gw2), jnp.float32)], axis=0)
    er2_main = jnp.concatenate(
        [er2, jnp.full((n_nodes, _L - h2), -60.0, jnp.float32)], axis=1)
    er2_slab = jnp.concatenate(
        [er2_main, jnp.zeros((np_rows - n_nodes, _L), jnp.float32)], axis=0)
    for cc in range(_NC):
        er2_ref[cc] = er2_slab
    cm2_row = jnp.concatenate(
        [cm2, jnp.zeros((1, _L - h2), jnp.float32)], axis=1)
    cm2_ref[...] = jnp.concatenate([cm2_row] * _NC, axis=0)


def _fin_body(o2_ref, out_ref):
    a = o2_ref[0] + o2_ref[1]
    n, d = out_ref.shape
    num = a[:n, :d]
    den = a[:n, d:d + 1]
    out_ref[...] = num / (den + 1e-9)


def _edge_pass(gtab, ertab, cmtab, srcp, dstp, n_nodes, hh, dim,
               split_heads):
    rt = srcp.shape[0]
    nwork = _NS if split_heads else _NC * _NS
    rti = rt // nwork
    bch = 4
    nsuper = rti // bch
    assert rti % bch == 0
    np_rows = -(-n_nodes // (_NS * 8)) * (_NS * 8)
    acc_rows = np_rows
    rpt = acc_rows // _NS
    gw = dim * hh + _L
    fd = dim * hh
    zrows = 8
    assert rpt % zrows == 0

    mesh = plsc.VectorSubcoreMesh(core_axis_name="c", subcore_axis_name="s",
                                  num_cores=_NC, num_subcores=_NS)

    def body(g_hbm, er_hbm, cm_hbm, src_hbm, dst_hbm, o_hbm,
             srcall, dstall, gb, eb, zbuf, cmbuf, ershr, *rest):
        if split_heads:
            ashr = rest[0]
        else:
            gshr, ashr = rest
        cc = lax.axis_index("c")
        ss = lax.axis_index("s")
        row0 = ss * rpt
        wid = ss if split_heads else ss * _NC + cc
        pltpu.sync_copy(cm_hbm.at[cc], cmbuf)
        pltpu.sync_copy(src_hbm.at[pl.ds(wid * rti, rti)], srcall)
        pltpu.sync_copy(dst_hbm.at[pl.ds(wid * rti, rti)], dstall)

        if split_heads:
            rowoff = cc * n_nodes

            def trow(r, carry):
                for g in range(8):
                    srcall[r, pl.ds(_L * g, _L)] = (
                        srcall[r, pl.ds(_L * g, _L)] + rowoff)
                return carry

            lax.fori_loop(0, rti, trow, 0, unroll=4)

        zero = jnp.zeros((_L,), jnp.float32)
        for r in range(zrows):
            for j in range(gw // _L):
                zbuf[r, pl.ds(_L * j, _L)] = zero
        for b in range(rpt // zrows):
            pltpu.sync_copy(zbuf, ashr.at[pl.ds(row0 + zrows * b, zrows)])
        erpt = np_rows // _NS
        er0 = ss * erpt
        pltpu.sync_copy(er_hbm.at[cc, pl.ds(er0, erpt)],
                        ershr.at[pl.ds(er0, erpt)])
        if not split_heads:
            pltpu.sync_copy(g_hbm.at[pl.ds(er0, erpt)],
                            gshr.at[pl.ds(er0, erpt)])
        plsc.subcore_barrier()

        cmv = cmbuf[:]
        gsrc = g_hbm if split_heads else gshr

        def super_body(k, carry):
            pltpu.sync_copy(gsrc.at[srcall.at[k]], gb)
            pltpu.sync_copy(ershr.at[dstall.at[k]], eb)

            def ebody(i, carry2):
                elv = gb[i, pl.ds(fd, _L)]
                erv = eb[i, :]
                s = elv + erv
                e = jnp.where(s > 0, s, 0.2 * s) - cmv
                ee = jnp.exp(e)
                for h in range(hh):
                    gb[i, pl.ds(dim * h, _L)] = (
                        gb[i, pl.ds(dim * h, _L)] * ee[h])
                gb[i, pl.ds(fd, _L)] = ee
                return carry2

            lax.fori_loop(0, 128, ebody, 0, unroll=4)

            pltpu.sync_copy(gb, ashr.at[dstall.at[k]], add=True)
            return carry

        lax.fori_loop(0, rti, super_body, 0)

        plsc.subcore_barrier()
        pltpu.sync_copy(ashr.at[pl.ds(row0, rpt)],
                        o_hbm.at[cc, pl.ds(row0, rpt)])

    scratch = [
        pltpu.VMEM((rti, 128), jnp.int32),
        pltpu.VMEM((rti, 128), jnp.int32),
        pltpu.VMEM((128, gw), jnp.float32),
        pltpu.VMEM((128, _L), jnp.float32),
        pltpu.VMEM((zrows, gw), jnp.float32),
        pltpu.VMEM((_L,), jnp.float32),
        pltpu.VMEM_SHARED((np_rows, _L), jnp.float32),
    ] + ([] if split_heads else [
        pltpu.VMEM_SHARED((np_rows, gw), jnp.float32),
    ]) + [
        pltpu.VMEM_SHARED((acc_rows, gw), jnp.float32),
    ]
    out_type = jax.ShapeDtypeStruct((_NC, acc_rows, gw), jnp.float32)
    return pl.kernel(
        body, out_type=out_type, mesh=mesh, scratch_types=scratch,
        compiler_params=pltpu.CompilerParams(use_tc_tiling_on_sc=False),
    )(gtab, ertab, cmtab, srcp, dstp)


def kernel(x, edge_index, W1, attn_l1, attn_r1, W2, attn_l2, attn_r2):
    n, _ = x.shape
    h1, hid = attn_l1.shape
    h2, out_d = attn_l2.shape
    src = edge_index[0].astype(jnp.int32)
    dst = edge_index[1].astype(jnp.int32)

    e = src.shape[0]
    e_pad = -(-e // (_NS * 128 * 8)) * (_NS * 128 * 8)
    if e_pad > e:
        ii = jnp.arange(e_pad - e, dtype=jnp.int32)
        src = jnp.concatenate([src, ii % 1024])
        dst = jnp.concatenate([dst, n + (ii % 8)])
    src2d = src.reshape(e_pad // 128, 128)
    dst2d = dst.reshape(e_pad // 128, 128)

    al1 = (jnp.eye(h1, dtype=jnp.float32)[:, None, :]
           * attn_l1[:, :, None]).reshape(h1 * hid, h1)
    ar1 = (jnp.eye(h1, dtype=jnp.float32)[:, None, :]
           * attn_r1[:, :, None]).reshape(h1 * hid, h1)
    al2 = (jnp.eye(h2, dtype=jnp.float32)[:, None, :]
           * attn_l2[:, :, None]).reshape(h2 * out_d, h2)
    ar2 = (jnp.eye(h2, dtype=jnp.float32)[:, None, :]
           * attn_r2[:, :, None]).reshape(h2 * out_d, h2)
    s1 = jnp.repeat(jnp.eye(h1, dtype=jnp.float32), hid, axis=1)

    np_rows = -(-n // (_NS * 8)) * (_NS * 8)
    gw1 = (h1 // _NC) * hid + _L
    g1, er1, cm1 = pl.pallas_call(
        _prep1_body,
        out_shape=[jax.ShapeDtypeStruct((_NC * n, gw1), jnp.float32),
                   jax.ShapeDtypeStruct((_NC, np_rows, _L), jnp.float32),
                   jax.ShapeDtypeStruct((_NC, _L), jnp.float32)],
    )(x, W1, al1, ar1)

    o1 = _edge_pass(g1, er1, cm1, src2d, dst2d, n, h1 // _NC, hid, True)

    g2, er2, cm2 = pl.pallas_call(
        functools.partial(_mid_body, n),
        out_shape=[jax.ShapeDtypeStruct((np_rows, h2 * out_d + _L),
                                        jnp.float32),
                   jax.ShapeDtypeStruct((_NC, np_rows, _L), jnp.float32),
                   jax.ShapeDtypeStruct((_NC, _L), jnp.float32)],
    )(o1, W2, al2, ar2, s1)

    o2 = _edge_pass(g2, er2, cm2, src2d, dst2d, n, h2, out_d, False)

    out = pl.pallas_call(
        _fin_body,
        out_shape=jax.ShapeDtypeStruct((n, out_d), jnp.float32),
    )(o2)
    return out


# speedup vs baseline: 1.6364x; 1.0745x over previous
import functools

import jax
import jax.numpy as jnp
from jax import lax
from jax.experimental import pallas as pl
from jax.experimental.pallas import tpu as pltpu
from jax.experimental.pallas import tpu_sc as plsc

_NC = 2
_NS = 16
_L = 16


def _prep1_body(x_ref, w_ref, al_ref, ar_ref, g_ref, er_ref, cm_ref):
    feat = jnp.dot(x_ref[...], w_ref[...], preferred_element_type=jnp.float32)
    el = jnp.dot(feat, al_ref[...], preferred_element_type=jnp.float32)
    er = jnp.dot(feat, ar_ref[...], preferred_element_type=jnp.float32)
    c = jnp.max(el, axis=0, keepdims=True) + jnp.max(er, axis=0, keepdims=True)
    cm = jnp.where(c > 0, c, 0.2 * c)
    n, h = el.shape
    fw = feat.shape[1] // _NC
    hh = h // _NC
    halves_g = []
    halves_e = []
    halves_c = []
    for cc in range(_NC):
        halves_g.append(jnp.concatenate(
            [feat[:, cc * fw:(cc + 1) * fw],
             el[:, cc * hh:(cc + 1) * hh],
             jnp.zeros((n, _L - hh), jnp.float32)], axis=1))
        halves_e.append(jnp.concatenate(
            [er[:, cc * hh:(cc + 1) * hh],
             jnp.full((n, _L - hh), -60.0, jnp.float32)], axis=1))
        halves_c.append(jnp.concatenate(
            [cm[:, cc * hh:(cc + 1) * hh],
             jnp.zeros((1, _L - hh), jnp.float32)], axis=1))
    g_ref[...] = jnp.concatenate(halves_g, axis=0)
    np_rows = er_ref.shape[1]
    for cc in range(_NC):
        er_ref[cc] = jnp.concatenate(
            [halves_e[cc],
             jnp.zeros((np_rows - n, _L), jnp.float32)], axis=0)
    cm_ref[...] = jnp.concatenate(halves_c, axis=0)


def _mid_body(n_nodes, o1_ref, w2_ref, al2_ref, ar2_ref, s1_ref,
              g2_ref, er2_ref, cm2_ref):
    d = w2_ref.shape[0]
    h = s1_ref.shape[0]
    fw = d // _NC
    hh = h // _NC
    a0 = o1_ref[0]
    a1 = o1_ref[1]
    num = jnp.concatenate([a0[:n_nodes, :fw], a1[:n_nodes, :fw]], axis=1)
    den = jnp.concatenate([a0[:n_nodes, fw:fw + hh],
                           a1[:n_nodes, fw:fw + hh]], axis=1)
    den_w = jnp.dot(den, s1_ref[...], preferred_element_type=jnp.float32)
    q = num / (den_w + 1e-9)
    h1 = jnp.where(q > 0, q, jnp.exp(q) - 1.0)
    feat2 = jnp.dot(h1, w2_ref[...], preferred_element_type=jnp.float32)
    d2 = feat2.shape[1]
    el2 = jnp.dot(feat2, al2_ref[...], preferred_element_type=jnp.float32)
    er2 = jnp.dot(feat2, ar2_ref[...], preferred_element_type=jnp.float32)
    c2 = jnp.max(el2, axis=0, keepdims=True) + jnp.max(er2, axis=0,
                                                       keepdims=True)
    cm2 = jnp.where(c2 > 0, c2, 0.2 * c2)
    h2 = al2_ref.shape[1]
    gw2 = g2_ref.shape[1]
    np_rows = er2_ref.shape[1]
    g2_main = jnp.concatenate(
        [feat2, el2, jnp.zeros((n_nodes, gw2 - d2 - h2), jnp.float32)],
        axis=1)
    g2_ref[...] = jnp.concatenate(
        [g2_main, jnp.zeros((np_rows - n_nodes, gw2), jnp.float32)], axis=0)
    er2_main = jnp.concatenate(
        [er2, jnp.full((n_nodes, _L - h2), -60.0, jnp.float32)], axis=1)
    er2_slab = jnp.concatenate(
        [er2_main, jnp.zeros((np_rows - n_nodes, _L), jnp.float32)], axis=0)
    for cc in range(_NC):
        er2_ref[cc] = er2_slab
    cm2_row = jnp.concatenate(
        [cm2, jnp.zeros((1, _L - h2), jnp.float32)], axis=1)
    cm2_ref[...] = jnp.concatenate([cm2_row] * _NC, axis=0)


def _fin_body(o2_ref, out_ref):
    a = o2_ref[0] + o2_ref[1]
    n, d = out_ref.shape
    num = a[:n, :d]
    den = a[:n, d:d + 1]
    out_ref[...] = num / (den + 1e-9)


def _edge_pass(gtab, ertab, cmtab, srcp, dstp, n_nodes, hh, dim,
               split_heads):
    rt, cw = srcp.shape
    nwork = _NS if split_heads else _NC * _NS
    rti = rt // nwork
    bch = 4
    nsuper = rti // bch
    assert rti % bch == 0
    np_rows = -(-n_nodes // (_NS * 8)) * (_NS * 8)
    acc_rows = np_rows
    rpt = acc_rows // _NS
    gw = dim * hh + _L
    fd = dim * hh
    zrows = 8
    assert rpt % zrows == 0

    mesh = plsc.VectorSubcoreMesh(core_axis_name="c", subcore_axis_name="s",
                                  num_cores=_NC, num_subcores=_NS)

    def body(g_hbm, er_hbm, cm_hbm, src_hbm, dst_hbm, o_hbm,
             srcall, dstall, gb, eb, zbuf, cmbuf, ershr, *rest):
        if split_heads:
            ashr = rest[0]
        else:
            gshr, ashr = rest
        cc = lax.axis_index("c")
        ss = lax.axis_index("s")
        row0 = ss * rpt
        wid = ss if split_heads else ss * _NC + cc
        pltpu.sync_copy(cm_hbm.at[cc], cmbuf)
        pltpu.sync_copy(src_hbm.at[pl.ds(wid * rti, rti)], srcall)
        pltpu.sync_copy(dst_hbm.at[pl.ds(wid * rti, rti)], dstall)

        if split_heads:
            rowoff = cc * n_nodes

            def trow(r, carry):
                for g in range(cw // _L):
                    srcall[r, pl.ds(_L * g, _L)] = (
                        srcall[r, pl.ds(_L * g, _L)] + rowoff)
                return carry

            lax.fori_loop(0, rti, trow, 0, unroll=4)

        zero = jnp.zeros((_L,), jnp.float32)
        for r in range(zrows):
            for j in range(gw // _L):
                zbuf[r, pl.ds(_L * j, _L)] = zero
        for b in range(rpt // zrows):
            pltpu.sync_copy(zbuf, ashr.at[pl.ds(row0 + zrows * b, zrows)])
        erpt = np_rows // _NS
        er0 = ss * erpt
        pltpu.sync_copy(er_hbm.at[cc, pl.ds(er0, erpt)],
                        ershr.at[pl.ds(er0, erpt)])
        if not split_heads:
            pltpu.sync_copy(g_hbm.at[pl.ds(er0, erpt)],
                            gshr.at[pl.ds(er0, erpt)])
        plsc.subcore_barrier()

        cmv = cmbuf[:]
        gsrc = g_hbm if split_heads else gshr

        def super_body(k, carry):
            pltpu.sync_copy(gsrc.at[srcall.at[k]], gb)
            pltpu.sync_copy(ershr.at[dstall.at[k]], eb)

            def ebody(i, carry2):
                elv = gb[i, pl.ds(fd, _L)]
                erv = eb[i, :]
                s = elv + erv
                e = jnp.where(s > 0, s, 0.2 * s) - cmv
                ee = jnp.exp(e)
                for h in range(hh):
                    gb[i, pl.ds(dim * h, _L)] = (
                        gb[i, pl.ds(dim * h, _L)] * ee[h])
                gb[i, pl.ds(fd, _L)] = ee
                return carry2

            lax.fori_loop(0, cw, ebody, 0, unroll=4)

            pltpu.sync_copy(gb, ashr.at[dstall.at[k]], add=True)
            return carry

        lax.fori_loop(0, rti, super_body, 0)

        plsc.subcore_barrier()
        pltpu.sync_copy(ashr.at[pl.ds(row0, rpt)],
                        o_hbm.at[cc, pl.ds(row0, rpt)])

    scratch = [
        pltpu.VMEM((rti, cw), jnp.int32),
        pltpu.VMEM((rti, cw), jnp.int32),
        pltpu.VMEM((cw, gw), jnp.float32),
        pltpu.VMEM((cw, _L), jnp.float32),
        pltpu.VMEM((zrows, gw), jnp.float32),
        pltpu.VMEM((_L,), jnp.float32),
        pltpu.VMEM_SHARED((np_rows, _L), jnp.float32),
    ] + ([] if split_heads else [
        pltpu.VMEM_SHARED((np_rows, gw), jnp.float32),
    ]) + [
        pltpu.VMEM_SHARED((acc_rows, gw), jnp.float32),
    ]
    out_type = jax.ShapeDtypeStruct((_NC, acc_rows, gw), jnp.float32)
    return pl.kernel(
        body, out_type=out_type, mesh=mesh, scratch_types=scratch,
        compiler_params=pltpu.CompilerParams(use_tc_tiling_on_sc=False),
    )(gtab, ertab, cmtab, srcp, dstp)


def kernel(x, edge_index, W1, attn_l1, attn_r1, W2, attn_l2, attn_r2):
    n, _ = x.shape
    h1, hid = attn_l1.shape
    h2, out_d = attn_l2.shape
    src = edge_index[0].astype(jnp.int32)
    dst = edge_index[1].astype(jnp.int32)

    e = src.shape[0]
    e_pad = -(-e // (_NS * 128 * 8)) * (_NS * 128 * 8)
    if e_pad > e:
        ii = jnp.arange(e_pad - e, dtype=jnp.int32)
        src = jnp.concatenate([src, ii % 1024])
        dst = jnp.concatenate([dst, n + (ii % 8)])
    src2d = src.reshape(e_pad // 256, 256)
    dst2d = dst.reshape(e_pad // 256, 256)

    al1 = (jnp.eye(h1, dtype=jnp.float32)[:, None, :]
           * attn_l1[:, :, None]).reshape(h1 * hid, h1)
    ar1 = (jnp.eye(h1, dtype=jnp.float32)[:, None, :]
           * attn_r1[:, :, None]).reshape(h1 * hid, h1)
    al2 = (jnp.eye(h2, dtype=jnp.float32)[:, None, :]
           * attn_l2[:, :, None]).reshape(h2 * out_d, h2)
    ar2 = (jnp.eye(h2, dtype=jnp.float32)[:, None, :]
           * attn_r2[:, :, None]).reshape(h2 * out_d, h2)
    s1 = jnp.repeat(jnp.eye(h1, dtype=jnp.float32), hid, axis=1)

    np_rows = -(-n // (_NS * 8)) * (_NS * 8)
    gw1 = (h1 // _NC) * hid + _L
    g1, er1, cm1 = pl.pallas_call(
        _prep1_body,
        out_shape=[jax.ShapeDtypeStruct((_NC * n, gw1), jnp.float32),
                   jax.ShapeDtypeStruct((_NC, np_rows, _L), jnp.float32),
                   jax.ShapeDtypeStruct((_NC, _L), jnp.float32)],
    )(x, W1, al1, ar1)

    o1 = _edge_pass(g1, er1, cm1, src2d, dst2d, n, h1 // _NC, hid, True)

    g2, er2, cm2 = pl.pallas_call(
        functools.partial(_mid_body, n),
        out_shape=[jax.ShapeDtypeStruct((np_rows, h2 * out_d + _L),
                                        jnp.float32),
                   jax.ShapeDtypeStruct((_NC, np_rows, _L), jnp.float32),
                   jax.ShapeDtypeStruct((_NC, _L), jnp.float32)],
    )(o1, W2, al2, ar2, s1)

    o2 = _edge_pass(g2, er2, cm2, src2d, dst2d, n, h2, out_d, False)

    out = pl.pallas_call(
        _fin_body,
        out_shape=jax.ShapeDtypeStruct((n, out_d), jnp.float32),
    )(o2)
    return out
